# Initial kernel scaffold; baseline (speedup 1.0000x reference)
import jax
import jax.numpy as jnp
from jax.experimental import pallas as pl


def kernel(x, edge_index, w1_rel, b1, w1_root, g1, be1, w2_rel, b2, w2_root, g2, be2, w_fc, b_fc, g3, be3, w3_rel, b3, w3_root, g4, be4, wmu_rel, bmu, wmu_root, wls_rel, bls, wls_root):
    raise NotImplementedError("write your pallas kernel here")


# speedup vs baseline: 3.6773x; 3.6773x over previous
import functools

import jax
import jax.numpy as jnp
from jax import lax
from jax.experimental import pallas as pl
from jax.experimental.pallas import tpu as pltpu
from jax.experimental.pallas import tpu_sc as plsc

N = 10000
E = 320000
D_IN = 128
H = 256
L = 64

NC = 2
NS = 16
DUMP_ROWS = 624
LAST_ROWS = N - 15 * DUMP_ROWS
EDGES_PER_TILE = E // NS
CHUNK = 80
NCHUNK = EDGES_PER_TILE // CHUNK
ZROWS = 16

_MESH = plsc.VectorSubcoreMesh(
    core_axis_name="c", subcore_axis_name="s", num_cores=NC, num_subcores=NS)


def _make_segsum(dh):
  @functools.partial(
      pl.kernel,
      out_type=(jax.ShapeDtypeStruct((N, dh), jnp.float32),
                jax.ShapeDtypeStruct((N, dh), jnp.float32)),
      mesh=_MESH,
      scratch_types=[
          pltpu.VMEM((CHUNK,), jnp.int32),
          pltpu.VMEM((CHUNK,), jnp.int32),
          pltpu.VMEM((CHUNK, dh), jnp.float32),
          pltpu.VMEM((ZROWS, dh), jnp.float32),
          pltpu.VMEM_SHARED((N, dh), jnp.float32),
          pltpu.SemaphoreType.DMA,
      ])
  def segsum(t0, t1, src_hbm, dst_hbm, o0, o1,
             src_v, dst_v, rows_v, zbuf, accum, sem):
    cid = lax.axis_index("c")
    sid = lax.axis_index("s")

    zeros = jnp.zeros((16,), jnp.float32)

    def zrow(r, carry):
      for j in range(dh // 16):
        zbuf[r, pl.ds(j * 16, 16)] = zeros
      return carry
    lax.fori_loop(0, ZROWS, zrow, 0)

    row0 = sid * DUMP_ROWS
    nzfill = jnp.where(sid == NS - 1, LAST_ROWS // ZROWS, DUMP_ROWS // ZROWS)

    def zfill(k, carry):
      pltpu.sync_copy(zbuf, accum.at[pl.ds(row0 + k * ZROWS, ZROWS)])
      return carry
    lax.fori_loop(0, nzfill, zfill, 0)

    plsc.subcore_barrier()

    def chunk(i, carry):
      base = sid * EDGES_PER_TILE + i * CHUNK
      pltpu.sync_copy(src_hbm.at[pl.ds(base, CHUNK)], src_v)
      pltpu.sync_copy(dst_hbm.at[pl.ds(base, CHUNK)], dst_v)

      @pl.when(cid == 0)
      def _():
        pltpu.async_copy(t0.at[src_v], rows_v, sem).wait()

      @pl.when(cid == 1)
      def _():
        pltpu.async_copy(t1.at[src_v], rows_v, sem).wait()

      pltpu.sync_copy(rows_v, accum.at[dst_v], add=True)
      return carry
    lax.fori_loop(0, NCHUNK, chunk, 0)

    plsc.subcore_barrier()

    last = sid == NS - 1
    for c, o in ((0, o0), (1, o1)):
      @pl.when(jnp.logical_and(cid == c, jnp.logical_not(last)))
      def _(o=o):
        pltpu.sync_copy(accum.at[pl.ds(row0, DUMP_ROWS)],
                        o.at[pl.ds(row0, DUMP_ROWS)])

      @pl.when(jnp.logical_and(cid == c, last))
      def _(o=o):
        pltpu.sync_copy(accum.at[pl.ds(row0, LAST_ROWS)],
                        o.at[pl.ds(row0, LAST_ROWS)])

  return segsum


def _make_segsum_edge(d):
  epc = E // NC
  ept = epc // NS
  nchunk = ept // CHUNK

  @functools.partial(
      pl.kernel,
      out_type=(jax.ShapeDtypeStruct((N, d), jnp.float32),
                jax.ShapeDtypeStruct((N, d), jnp.float32)),
      mesh=_MESH,
      scratch_types=[
          pltpu.VMEM((CHUNK,), jnp.int32),
          pltpu.VMEM((CHUNK,), jnp.int32),
          pltpu.VMEM((CHUNK, d), jnp.float32),
          pltpu.VMEM((ZROWS, d), jnp.float32),
          pltpu.VMEM_SHARED((N, d), jnp.float32),
          pltpu.SemaphoreType.DMA,
      ])
  def segsum(table, src_hbm, dst_hbm, o0, o1,
             src_v, dst_v, rows_v, zbuf, accum, sem):
    cid = lax.axis_index("c")
    sid = lax.axis_index("s")

    zeros = jnp.zeros((16,), jnp.float32)

    def zrow(r, carry):
      for j in range(d // 16):
        zbuf[r, pl.ds(j * 16, 16)] = zeros
      return carry
    lax.fori_loop(0, ZROWS, zrow, 0)

    row0 = sid * DUMP_ROWS
    nzfill = jnp.where(sid == NS - 1, LAST_ROWS // ZROWS, DUMP_ROWS // ZROWS)

    def zfill(k, carry):
      pltpu.sync_copy(zbuf, accum.at[pl.ds(row0 + k * ZROWS, ZROWS)])
      return carry
    lax.fori_loop(0, nzfill, zfill, 0)

    plsc.subcore_barrier()

    def chunk(i, carry):
      base = cid * epc + sid * ept + i * CHUNK
      pltpu.sync_copy(src_hbm.at[pl.ds(base, CHUNK)], src_v)
      pltpu.sync_copy(dst_hbm.at[pl.ds(base, CHUNK)], dst_v)
      pltpu.async_copy(table.at[src_v], rows_v, sem).wait()
      pltpu.sync_copy(rows_v, accum.at[dst_v], add=True)
      return carry
    lax.fori_loop(0, nchunk, chunk, 0)

    plsc.subcore_barrier()

    last = sid == NS - 1
    for c, o in ((0, o0), (1, o1)):
      @pl.when(jnp.logical_and(cid == c, jnp.logical_not(last)))
      def _(o=o):
        pltpu.sync_copy(accum.at[pl.ds(row0, DUMP_ROWS)],
                        o.at[pl.ds(row0, DUMP_ROWS)])

      @pl.when(jnp.logical_and(cid == c, last))
      def _(o=o):
        pltpu.sync_copy(accum.at[pl.ds(row0, LAST_ROWS)],
                        o.at[pl.ds(row0, LAST_ROWS)])

  return segsum


_segsum_edge128 = _make_segsum_edge(D_IN)
_segsum128 = _make_segsum(H // 2)

R = 1000
GRID = N // R

_f32 = jnp.float32
_DOT = dict(preferred_element_type=_f32, precision=lax.Precision.HIGHEST)


def _ln(h, g, be):
  m = jnp.mean(h, axis=-1, keepdims=True)
  hc = h - m
  v = jnp.mean(hc * hc, axis=-1, keepdims=True)
  return hc * lax.rsqrt(v + 1e-5) * g + be


def _full(shape):
  return pl.BlockSpec(shape, lambda i: (0,) * len(shape))


def _rows(w):
  return pl.BlockSpec((R, w), lambda i: (i, 0))


def _dense1_body(a0, a1, x, wrelT, b, wrootT, g, be, y0, y1):
  aggr = a0[...] + a1[...]
  h = (jnp.dot(aggr, wrelT[...], **_DOT)
       + jnp.dot(x[...], wrootT[...], **_DOT) + b[...])
  o = _ln(jnp.maximum(h, 0.0), g[...], be[...])
  y0[...] = o[:, :H // 2]
  y1[...] = o[:, H // 2:]


_dense1 = pl.pallas_call(
    _dense1_body,
    grid=(GRID,),
    in_specs=[_rows(128), _rows(128), _rows(128), _full((D_IN, H)),
              _full((1, H)), _full((D_IN, H)), _full((1, H)), _full((1, H))],
    out_specs=(_rows(128), _rows(128)),
    out_shape=(jax.ShapeDtypeStruct((N, 128), _f32),
               jax.ShapeDtypeStruct((N, 128), _f32)),
)


def _dense2_body(a0, a1, x0, x1, wrelT, b, wrootT, g, be,
                 wfcT, bfc, g3, be3, y0, y1):
  x1f = jnp.concatenate([x0[...], x1[...]], axis=1)
  aggr = jnp.concatenate([a0[...], a1[...]], axis=1)
  h = (jnp.dot(aggr, wrelT[...], **_DOT)
       + jnp.dot(x1f, wrootT[...], **_DOT) + b[...])
  x2 = _ln(jnp.maximum(h, 0.0), g[...], be[...])
  h3 = jnp.dot(x1f + x2, wfcT[...], **_DOT) + bfc[...]
  o = _ln(jnp.maximum(h3, 0.0), g3[...], be3[...])
  y0[...] = o[:, :H // 2]
  y1[...] = o[:, H // 2:]


_dense2 = pl.pallas_call(
    _dense2_body,
    grid=(GRID,),
    in_specs=[_rows(128), _rows(128), _rows(128), _rows(128),
              _full((H, H)), _full((1, H)), _full((H, H)), _full((1, H)),
              _full((1, H)), _full((H, H)), _full((1, H)), _full((1, H)),
              _full((1, H))],
    out_specs=(_rows(128), _rows(128)),
    out_shape=(jax.ShapeDtypeStruct((N, 128), _f32),
               jax.ShapeDtypeStruct((N, 128), _f32)),
)


def _dense3_body(a0, a1, x0, x1, wrelT, b, wrootT, g, be, y0, y1):
  xf = jnp.concatenate([x0[...], x1[...]], axis=1)
  aggr = jnp.concatenate([a0[...], a1[...]], axis=1)
  h = (jnp.dot(aggr, wrelT[...], **_DOT)
       + jnp.dot(xf, wrootT[...], **_DOT) + b[...])
  o = _ln(jnp.maximum(h, 0.0), g[...], be[...])
  y0[...] = o[:, :H // 2]
  y1[...] = o[:, H // 2:]


_dense3 = pl.pallas_call(
    _dense3_body,
    grid=(GRID,),
    in_specs=[_rows(128), _rows(128), _rows(128), _rows(128),
              _full((H, H)), _full((1, H)), _full((H, H)), _full((1, H)),
              _full((1, H))],
    out_specs=(_rows(128), _rows(128)),
    out_shape=(jax.ShapeDtypeStruct((N, 128), _f32),
               jax.ShapeDtypeStruct((N, 128), _f32)),
)


def _dense4_body(a0, a1, x0, x1, wmu_relT, bmu, wmu_rootT,
                 wls_relT, bls, wls_rootT, mu, ls):
  xf = jnp.concatenate([x0[...], x1[...]], axis=1)
  aggr = jnp.concatenate([a0[...], a1[...]], axis=1)
  mu[...] = (jnp.dot(aggr, wmu_relT[...], **_DOT)
             + jnp.dot(xf, wmu_rootT[...], **_DOT) + bmu[...])
  ls[...] = (jnp.dot(aggr, wls_relT[...], **_DOT)
             + jnp.dot(xf, wls_rootT[...], **_DOT) + bls[...])


_dense4 = pl.pallas_call(
    _dense4_body,
    grid=(GRID,),
    in_specs=[_rows(128), _rows(128), _rows(128), _rows(128),
              _full((H, L)), _full((1, L)), _full((H, L)),
              _full((H, L)), _full((1, L)), _full((H, L))],
    out_specs=(_rows(L), _rows(L)),
    out_shape=(jax.ShapeDtypeStruct((N, L), _f32),
               jax.ShapeDtypeStruct((N, L), _f32)),
)


def kernel(x, edge_index, w1_rel, b1, w1_root, g1, be1, w2_rel, b2, w2_root,
           g2, be2, w_fc, b_fc, g3, be3, w3_rel, b3, w3_root, g4, be4,
           wmu_rel, bmu, wmu_root, wls_rel, bls, wls_root):
  src = edge_index[0]
  dst = edge_index[1]
  row = lambda v: v.reshape(1, -1)

  a1_0, a1_1 = _segsum_edge128(x, src, dst)
  x1_0, x1_1 = _dense1(a1_0, a1_1, x, w1_rel.T, row(b1), w1_root.T,
                       row(g1), row(be1))
  a2_0, a2_1 = _segsum128(x1_0, x1_1, src, dst)
  x3_0, x3_1 = _dense2(a2_0, a2_1, x1_0, x1_1, w2_rel.T, row(b2), w2_root.T,
                       row(g2), row(be2), w_fc.T, row(b_fc), row(g3), row(be3))
  a3_0, a3_1 = _segsum128(x3_0, x3_1, src, dst)
  x4_0, x4_1 = _dense3(a3_0, a3_1, x3_0, x3_1, w3_rel.T, row(b3), w3_root.T,
                       row(g4), row(be4))
  a4_0, a4_1 = _segsum128(x4_0, x4_1, src, dst)
  mu, logstd = _dense4(a4_0, a4_1, x4_0, x4_1, wmu_rel.T, row(bmu),
                       wmu_root.T, wls_rel.T, row(bls), wls_root.T)
  return (mu, logstd)


# speedup vs baseline: 9.4070x; 2.5581x over previous
import functools

import jax
import jax.numpy as jnp
from jax import lax
from jax.experimental import pallas as pl
from jax.experimental.pallas import tpu as pltpu
from jax.experimental.pallas import tpu_sc as plsc

N = 10000
E = 320000
D_IN = 128
H = 256
L = 64

NC = 2
NS = 16
DUMP_ROWS = 624
LAST_ROWS = N - 15 * DUMP_ROWS
CHUNK = 125
NBUF = 2
KSLAB = 40
ZROWS = 16

_MESH = plsc.VectorSubcoreMesh(
    core_axis_name="c", subcore_axis_name="s", num_cores=NC, num_subcores=NS)


def _make_segsum(dh, feat_split):
  nch = E // CHUNK // NS // (1 if feat_split else NC)

  @functools.partial(
      pl.kernel,
      out_type=(jax.ShapeDtypeStruct((N, dh), jnp.float32),
                jax.ShapeDtypeStruct((N, dh), jnp.float32)),
      mesh=_MESH,
      scratch_types=[
          pltpu.VMEM((KSLAB, CHUNK), jnp.int32),
          pltpu.VMEM((KSLAB, CHUNK), jnp.int32),
          [pltpu.VMEM((CHUNK, dh), jnp.float32)] * NBUF,
          pltpu.VMEM((ZROWS, dh), jnp.float32),
          pltpu.VMEM_SHARED((N, dh), jnp.float32),
          [pltpu.SemaphoreType.DMA] * NBUF,
      ])
  def segsum(t0, t1, src2, dst2, o0, o1,
             src_all, dst_all, rows, zbuf, accum, sems):
    cid = lax.axis_index("c")
    sid = lax.axis_index("s")

    zeros = jnp.zeros((16,), jnp.float32)

    def zrow(r, carry):
      for j in range(dh // 16):
        zbuf[r, pl.ds(j * 16, 16)] = zeros
      return carry
    lax.fori_loop(0, ZROWS, zrow, 0)

    row0 = sid * DUMP_ROWS
    nzfill = jnp.where(sid == NS - 1, LAST_ROWS // ZROWS, DUMP_ROWS // ZROWS)

    def zfill(k, carry):
      pltpu.sync_copy(zbuf, accum.at[pl.ds(row0 + k * ZROWS, ZROWS)])
      return carry
    lax.fori_loop(0, nzfill, zfill, 0)

    ibase = (sid * nch) if feat_split else (cid * NS * nch + sid * nch)

    def load_slab(j0):
      pltpu.sync_copy(src2.at[pl.ds(ibase + j0, KSLAB)], src_all)
      pltpu.sync_copy(dst2.at[pl.ds(ibase + j0, KSLAB)], dst_all)

    plsc.subcore_barrier()

    def gather(jm, b, wait):
      def issue(t):
        d = pltpu.make_async_copy(t.at[src_all.at[jm]], rows[b], sems[b])
        if wait:
          d.wait()
        else:
          d.start()
      if feat_split:
        @pl.when(cid == 0)
        def _():
          issue(t0)

        @pl.when(cid == 1)
        def _():
          issue(t1)
      else:
        issue(t0)

    def slab_body(t, carry):
      load_slab(t * KSLAB)
      for b in range(NBUF):
        gather(b, b, wait=False)

      def chunk(jm, carry2):
        for b in range(NBUF):
          @pl.when(lax.rem(jm, NBUF) == b)
          def _(b=b):
            gather(jm, b, wait=True)
            pltpu.sync_copy(rows[b], accum.at[dst_all.at[jm]], add=True)

            @pl.when(jm + NBUF < KSLAB)
            def _():
              gather(jm + NBUF, b, wait=False)
        return carry2
      lax.fori_loop(0, KSLAB, chunk, 0)
      return carry
    lax.fori_loop(0, nch // KSLAB, slab_body, 0)

    plsc.subcore_barrier()

    last = sid == NS - 1
    for c, o in ((0, o0), (1, o1)):
      @pl.when(jnp.logical_and(cid == c, jnp.logical_not(last)))
      def _(o=o):
        pltpu.sync_copy(accum.at[pl.ds(row0, DUMP_ROWS)],
                        o.at[pl.ds(row0, DUMP_ROWS)])

      @pl.when(jnp.logical_and(cid == c, last))
      def _(o=o):
        pltpu.sync_copy(accum.at[pl.ds(row0, LAST_ROWS)],
                        o.at[pl.ds(row0, LAST_ROWS)])

  return segsum


_segsum_edge128 = _make_segsum(D_IN, feat_split=False)
_segsum128 = _make_segsum(H // 2, feat_split=True)

R = 1000
GRID = N // R

_f32 = jnp.float32
_DOT = dict(preferred_element_type=_f32, precision=lax.Precision.HIGHEST)


def _ln(h, g, be):
  m = jnp.mean(h, axis=-1, keepdims=True)
  hc = h - m
  v = jnp.mean(hc * hc, axis=-1, keepdims=True)
  return hc * lax.rsqrt(v + 1e-5) * g + be


def _full(shape):
  return pl.BlockSpec(shape, lambda i: (0,) * len(shape))


def _rows(w):
  return pl.BlockSpec((R, w), lambda i: (i, 0))


def _dense1_body(a0, a1, x, wrelT, b, wrootT, g, be, y0, y1):
  aggr = a0[...] + a1[...]
  h = (jnp.dot(aggr, wrelT[...], **_DOT)
       + jnp.dot(x[...], wrootT[...], **_DOT) + b[...])
  o = _ln(jnp.maximum(h, 0.0), g[...], be[...])
  y0[...] = o[:, :H // 2]
  y1[...] = o[:, H // 2:]


_dense1 = pl.pallas_call(
    _dense1_body,
    grid=(GRID,),
    in_specs=[_rows(128), _rows(128), _rows(128), _full((D_IN, H)),
              _full((1, H)), _full((D_IN, H)), _full((1, H)), _full((1, H))],
    out_specs=(_rows(128), _rows(128)),
    out_shape=(jax.ShapeDtypeStruct((N, 128), _f32),
               jax.ShapeDtypeStruct((N, 128), _f32)),
)


def _dense2_body(a0, a1, x0, x1, wrelT, b, wrootT, g, be,
                 wfcT, bfc, g3, be3, y0, y1):
  x1f = jnp.concatenate([x0[...], x1[...]], axis=1)
  aggr = jnp.concatenate([a0[...], a1[...]], axis=1)
  h = (jnp.dot(aggr, wrelT[...], **_DOT)
       + jnp.dot(x1f, wrootT[...], **_DOT) + b[...])
  x2 = _ln(jnp.maximum(h, 0.0), g[...], be[...])
  h3 = jnp.dot(x1f + x2, wfcT[...], **_DOT) + bfc[...]
  o = _ln(jnp.maximum(h3, 0.0), g3[...], be3[...])
  y0[...] = o[:, :H // 2]
  y1[...] = o[:, H // 2:]


_dense2 = pl.pallas_call(
    _dense2_body,
    grid=(GRID,),
    in_specs=[_rows(128), _rows(128), _rows(128), _rows(128),
              _full((H, H)), _full((1, H)), _full((H, H)), _full((1, H)),
              _full((1, H)), _full((H, H)), _full((1, H)), _full((1, H)),
              _full((1, H))],
    out_specs=(_rows(128), _rows(128)),
    out_shape=(jax.ShapeDtypeStruct((N, 128), _f32),
               jax.ShapeDtypeStruct((N, 128), _f32)),
)


def _dense3_body(a0, a1, x0, x1, wrelT, b, wrootT, g, be, y0, y1):
  xf = jnp.concatenate([x0[...], x1[...]], axis=1)
  aggr = jnp.concatenate([a0[...], a1[...]], axis=1)
  h = (jnp.dot(aggr, wrelT[...], **_DOT)
       + jnp.dot(xf, wrootT[...], **_DOT) + b[...])
  o = _ln(jnp.maximum(h, 0.0), g[...], be[...])
  y0[...] = o[:, :H // 2]
  y1[...] = o[:, H // 2:]


_dense3 = pl.pallas_call(
    _dense3_body,
    grid=(GRID,),
    in_specs=[_rows(128), _rows(128), _rows(128), _rows(128),
              _full((H, H)), _full((1, H)), _full((H, H)), _full((1, H)),
              _full((1, H))],
    out_specs=(_rows(128), _rows(128)),
    out_shape=(jax.ShapeDtypeStruct((N, 128), _f32),
               jax.ShapeDtypeStruct((N, 128), _f32)),
)


def _dense4_body(a0, a1, x0, x1, wmu_relT, bmu, wmu_rootT,
                 wls_relT, bls, wls_rootT, mu, ls):
  xf = jnp.concatenate([x0[...], x1[...]], axis=1)
  aggr = jnp.concatenate([a0[...], a1[...]], axis=1)
  mu[...] = (jnp.dot(aggr, wmu_relT[...], **_DOT)
             + jnp.dot(xf, wmu_rootT[...], **_DOT) + bmu[...])
  ls[...] = (jnp.dot(aggr, wls_relT[...], **_DOT)
             + jnp.dot(xf, wls_rootT[...], **_DOT) + bls[...])


_dense4 = pl.pallas_call(
    _dense4_body,
    grid=(GRID,),
    in_specs=[_rows(128), _rows(128), _rows(128), _rows(128),
              _full((H, L)), _full((1, L)), _full((H, L)),
              _full((H, L)), _full((1, L)), _full((H, L))],
    out_specs=(_rows(L), _rows(L)),
    out_shape=(jax.ShapeDtypeStruct((N, L), _f32),
               jax.ShapeDtypeStruct((N, L), _f32)),
)


def kernel(x, edge_index, w1_rel, b1, w1_root, g1, be1, w2_rel, b2, w2_root,
           g2, be2, w_fc, b_fc, g3, be3, w3_rel, b3, w3_root, g4, be4,
           wmu_rel, bmu, wmu_root, wls_rel, bls, wls_root):
  src = edge_index[0].reshape(E // CHUNK, CHUNK)
  dst = edge_index[1].reshape(E // CHUNK, CHUNK)
  row = lambda v: v.reshape(1, -1)

  a1_0, a1_1 = _segsum_edge128(x, x, src, dst)
  x1_0, x1_1 = _dense1(a1_0, a1_1, x, w1_rel.T, row(b1), w1_root.T,
                       row(g1), row(be1))
  a2_0, a2_1 = _segsum128(x1_0, x1_1, src, dst)
  x3_0, x3_1 = _dense2(a2_0, a2_1, x1_0, x1_1, w2_rel.T, row(b2), w2_root.T,
                       row(g2), row(be2), w_fc.T, row(b_fc), row(g3), row(be3))
  a3_0, a3_1 = _segsum128(x3_0, x3_1, src, dst)
  x4_0, x4_1 = _dense3(a3_0, a3_1, x3_0, x3_1, w3_rel.T, row(b3), w3_root.T,
                       row(g4), row(be4))
  a4_0, a4_1 = _segsum128(x4_0, x4_1, src, dst)
  mu, logstd = _dense4(a4_0, a4_1, x4_0, x4_1, wmu_rel.T, row(bmu),
                       wmu_root.T, wls_rel.T, row(bls), wls_root.T)
  return (mu, logstd)


# speedup vs baseline: 10.2051x; 1.0848x over previous
import functools

import jax
import jax.numpy as jnp
from jax import lax
from jax.experimental import pallas as pl
from jax.experimental.pallas import tpu as pltpu
from jax.experimental.pallas import tpu_sc as plsc

N = 10000
E = 320000
D_IN = 128
H = 256
L = 64

NC = 2
NS = 16
DUMP_ROWS = 624
LAST_ROWS = N - 15 * DUMP_ROWS
CHUNK = 125
NBUF = 2
KSLAB = 40
ZROWS = 16

_MESH = plsc.VectorSubcoreMesh(
    core_axis_name="c", subcore_axis_name="s", num_cores=NC, num_subcores=NS)


def _make_segsum(dh, feat_split):
  nch = E // CHUNK // NS // (1 if feat_split else NC)

  @functools.partial(
      pl.kernel,
      out_type=(jax.ShapeDtypeStruct((N, dh), jnp.float32),
                jax.ShapeDtypeStruct((N, dh), jnp.float32)),
      mesh=_MESH,
      scratch_types=[
          pltpu.VMEM((KSLAB, CHUNK), jnp.int32),
          pltpu.VMEM((KSLAB, CHUNK), jnp.int32),
          [pltpu.VMEM((CHUNK, dh), jnp.float32)] * NBUF,
          pltpu.VMEM((ZROWS, dh), jnp.float32),
          pltpu.VMEM_SHARED((N, dh), jnp.float32),
          [pltpu.SemaphoreType.DMA] * NBUF,
      ])
  def segsum(t0, t1, src2, dst2, o0, o1,
             src_all, dst_all, rows, zbuf, accum, sems):
    cid = lax.axis_index("c")
    sid = lax.axis_index("s")

    zeros = jnp.zeros((16,), jnp.float32)

    def zrow(r, carry):
      for j in range(dh // 16):
        zbuf[r, pl.ds(j * 16, 16)] = zeros
      return carry
    lax.fori_loop(0, ZROWS, zrow, 0)

    row0 = sid * DUMP_ROWS
    nzfill = jnp.where(sid == NS - 1, LAST_ROWS // ZROWS, DUMP_ROWS // ZROWS)

    def zfill(k, carry):
      pltpu.sync_copy(zbuf, accum.at[pl.ds(row0 + k * ZROWS, ZROWS)])
      return carry
    lax.fori_loop(0, nzfill, zfill, 0)

    ibase = (sid * nch) if feat_split else (cid * NS * nch + sid * nch)

    def load_slab(j0):
      pltpu.sync_copy(src2.at[pl.ds(ibase + j0, KSLAB)], src_all)
      pltpu.sync_copy(dst2.at[pl.ds(ibase + j0, KSLAB)], dst_all)

    plsc.subcore_barrier()

    def gather(jm, b, wait):
      def issue(t):
        d = pltpu.make_async_copy(t.at[src_all.at[jm]], rows[b], sems[b])
        if wait:
          d.wait()
        else:
          d.start()
      if feat_split:
        @pl.when(cid == 0)
        def _():
          issue(t0)

        @pl.when(cid == 1)
        def _():
          issue(t1)
      else:
        issue(t0)

    def slab_body(t, carry):
      load_slab(t * KSLAB)
      for b in range(NBUF):
        gather(b, b, wait=False)

      def chunk(jm, carry2):
        for b in range(NBUF):
          @pl.when(lax.rem(jm, NBUF) == b)
          def _(b=b):
            gather(jm, b, wait=True)
            pltpu.sync_copy(rows[b], accum.at[dst_all.at[jm]], add=True)

            @pl.when(jm + NBUF < KSLAB)
            def _():
              gather(jm + NBUF, b, wait=False)
        return carry2
      lax.fori_loop(0, KSLAB, chunk, 0)
      return carry
    lax.fori_loop(0, nch // KSLAB, slab_body, 0)

    plsc.subcore_barrier()

    last = sid == NS - 1
    for c, o in ((0, o0), (1, o1)):
      @pl.when(jnp.logical_and(cid == c, jnp.logical_not(last)))
      def _(o=o):
        pltpu.sync_copy(accum.at[pl.ds(row0, DUMP_ROWS)],
                        o.at[pl.ds(row0, DUMP_ROWS)])

      @pl.when(jnp.logical_and(cid == c, last))
      def _(o=o):
        pltpu.sync_copy(accum.at[pl.ds(row0, LAST_ROWS)],
                        o.at[pl.ds(row0, LAST_ROWS)])

  return segsum


_segsum_edge128 = _make_segsum(D_IN, feat_split=False)
_segsum128 = _make_segsum(H // 2, feat_split=True)

R = 1000
GRID = N // R

_f32 = jnp.float32
_DOT = dict(preferred_element_type=_f32, precision=lax.Precision.DEFAULT)


def _ln(h, g, be):
  m = jnp.mean(h, axis=-1, keepdims=True)
  hc = h - m
  v = jnp.mean(hc * hc, axis=-1, keepdims=True)
  return hc * lax.rsqrt(v + 1e-5) * g + be


def _full(shape):
  return pl.BlockSpec(shape, lambda i: (0,) * len(shape))


def _rows(w):
  return pl.BlockSpec((R, w), lambda i: (i, 0))


def _dense1_body(a0, a1, x, wrelT, b, wrootT, g, be, y0, y1):
  aggr = a0[...] + a1[...]
  h = (jnp.dot(aggr, wrelT[...], **_DOT)
       + jnp.dot(x[...], wrootT[...], **_DOT) + b[...])
  o = _ln(jnp.maximum(h, 0.0), g[...], be[...])
  y0[...] = o[:, :H // 2]
  y1[...] = o[:, H // 2:]


_dense1 = pl.pallas_call(
    _dense1_body,
    grid=(GRID,),
    in_specs=[_rows(128), _rows(128), _rows(128), _full((D_IN, H)),
              _full((1, H)), _full((D_IN, H)), _full((1, H)), _full((1, H))],
    out_specs=(_rows(128), _rows(128)),
    out_shape=(jax.ShapeDtypeStruct((N, 128), _f32),
               jax.ShapeDtypeStruct((N, 128), _f32)),
)


def _dense2_body(a0, a1, x0, x1, wrelT, b, wrootT, g, be,
                 wfcT, bfc, g3, be3, y0, y1):
  x1f = jnp.concatenate([x0[...], x1[...]], axis=1)
  aggr = jnp.concatenate([a0[...], a1[...]], axis=1)
  h = (jnp.dot(aggr, wrelT[...], **_DOT)
       + jnp.dot(x1f, wrootT[...], **_DOT) + b[...])
  x2 = _ln(jnp.maximum(h, 0.0), g[...], be[...])
  h3 = jnp.dot(x1f + x2, wfcT[...], **_DOT) + bfc[...]
  o = _ln(jnp.maximum(h3, 0.0), g3[...], be3[...])
  y0[...] = o[:, :H // 2]
  y1[...] = o[:, H // 2:]


_dense2 = pl.pallas_call(
    _dense2_body,
    grid=(GRID,),
    in_specs=[_rows(128), _rows(128), _rows(128), _rows(128),
              _full((H, H)), _full((1, H)), _full((H, H)), _full((1, H)),
              _full((1, H)), _full((H, H)), _full((1, H)), _full((1, H)),
              _full((1, H))],
    out_specs=(_rows(128), _rows(128)),
    out_shape=(jax.ShapeDtypeStruct((N, 128), _f32),
               jax.ShapeDtypeStruct((N, 128), _f32)),
)


def _dense3_body(a0, a1, x0, x1, wrelT, b, wrootT, g, be, y0, y1):
  xf = jnp.concatenate([x0[...], x1[...]], axis=1)
  aggr = jnp.concatenate([a0[...], a1[...]], axis=1)
  h = (jnp.dot(aggr, wrelT[...], **_DOT)
       + jnp.dot(xf, wrootT[...], **_DOT) + b[...])
  o = _ln(jnp.maximum(h, 0.0), g[...], be[...])
  y0[...] = o[:, :H // 2]
  y1[...] = o[:, H // 2:]


_dense3 = pl.pallas_call(
    _dense3_body,
    grid=(GRID,),
    in_specs=[_rows(128), _rows(128), _rows(128), _rows(128),
              _full((H, H)), _full((1, H)), _full((H, H)), _full((1, H)),
              _full((1, H))],
    out_specs=(_rows(128), _rows(128)),
    out_shape=(jax.ShapeDtypeStruct((N, 128), _f32),
               jax.ShapeDtypeStruct((N, 128), _f32)),
)


def _dense4_body(a0, a1, x0, x1, wmu_relT, bmu, wmu_rootT,
                 wls_relT, bls, wls_rootT, mu, ls):
  xf = jnp.concatenate([x0[...], x1[...]], axis=1)
  aggr = jnp.concatenate([a0[...], a1[...]], axis=1)
  mu[...] = (jnp.dot(aggr, wmu_relT[...], **_DOT)
             + jnp.dot(xf, wmu_rootT[...], **_DOT) + bmu[...])
  ls[...] = (jnp.dot(aggr, wls_relT[...], **_DOT)
             + jnp.dot(xf, wls_rootT[...], **_DOT) + bls[...])


_dense4 = pl.pallas_call(
    _dense4_body,
    grid=(GRID,),
    in_specs=[_rows(128), _rows(128), _rows(128), _rows(128),
              _full((H, L)), _full((1, L)), _full((H, L)),
              _full((H, L)), _full((1, L)), _full((H, L))],
    out_specs=(_rows(L), _rows(L)),
    out_shape=(jax.ShapeDtypeStruct((N, L), _f32),
               jax.ShapeDtypeStruct((N, L), _f32)),
)


def kernel(x, edge_index, w1_rel, b1, w1_root, g1, be1, w2_rel, b2, w2_root,
           g2, be2, w_fc, b_fc, g3, be3, w3_rel, b3, w3_root, g4, be4,
           wmu_rel, bmu, wmu_root, wls_rel, bls, wls_root):
  src = edge_index[0].reshape(E // CHUNK, CHUNK)
  dst = edge_index[1].reshape(E // CHUNK, CHUNK)
  row = lambda v: v.reshape(1, -1)

  a1_0, a1_1 = _segsum_edge128(x, x, src, dst)
  x1_0, x1_1 = _dense1(a1_0, a1_1, x, w1_rel.T, row(b1), w1_root.T,
                       row(g1), row(be1))
  a2_0, a2_1 = _segsum128(x1_0, x1_1, src, dst)
  x3_0, x3_1 = _dense2(a2_0, a2_1, x1_0, x1_1, w2_rel.T, row(b2), w2_root.T,
                       row(g2), row(be2), w_fc.T, row(b_fc), row(g3), row(be3))
  a3_0, a3_1 = _segsum128(x3_0, x3_1, src, dst)
  x4_0, x4_1 = _dense3(a3_0, a3_1, x3_0, x3_1, w3_rel.T, row(b3), w3_root.T,
                       row(g4), row(be4))
  a4_0, a4_1 = _segsum128(x4_0, x4_1, src, dst)
  mu, logstd = _dense4(a4_0, a4_1, x4_0, x4_1, wmu_rel.T, row(bmu),
                       wmu_root.T, wls_rel.T, row(bls), wls_root.T)
  return (mu, logstd)
